# Initial kernel scaffold; baseline (speedup 1.0000x reference)
import jax
import jax.numpy as jnp
from jax.experimental import pallas as pl


def kernel(x, edge_index, W1, b1, Wmu, bmu, Wlv, blv):
    raise NotImplementedError("write your pallas kernel here")


# speedup vs baseline: 12.4373x; 12.4373x over previous
import functools

import jax
import jax.numpy as jnp
from jax import lax
from jax.experimental import pallas as pl
from jax.experimental.pallas import tpu as pltpu
from jax.experimental.pallas import tpu_sc as plsc

NC = 2
NS = 16
CHUNK = 128

_MESH = plsc.VectorSubcoreMesh(core_axis_name="c", subcore_axis_name="s")


def _pad_nodes(N):
    return ((N + 8 * NS - 1) // (8 * NS)) * (8 * NS)


def _zero_fill(buf, rows, d):
    z = jnp.zeros((16,), jnp.float32)

    @pl.loop(0, rows)
    def _(i):
        @pl.loop(0, d // 16)
        def _(j):
            buf[i, pl.ds(j * 16, 16)] = z


@functools.lru_cache(maxsize=None)
def _deg_kernel(E, N):
    n_chunks = E // CHUNK
    Np = _pad_nodes(N)
    rpt = Np // NS

    @functools.partial(
        pl.kernel,
        mesh=_MESH,
        out_type=jax.ShapeDtypeStruct((NC, Np, 128), jnp.float32),
        scratch_types=[
            pltpu.VMEM((CHUNK,), jnp.int32),
            pltpu.VMEM((CHUNK, 128), jnp.float32),
            pltpu.VMEM((8, 128), jnp.float32),
            pltpu.VMEM_SHARED((Np, 128), jnp.float32),
        ],
    )
    def deg_k(dst_hbm, out_hbm, idx_v, ones_v, zbuf, acc_sh):
        c = lax.axis_index("c")
        s = lax.axis_index("s")
        wid = s * NC + c
        one = jnp.ones((16,), jnp.float32)

        @pl.loop(0, CHUNK)
        def _(i):
            @pl.loop(0, 8)
            def _(j):
                ones_v[i, pl.ds(j * 16, 16)] = one

        _zero_fill(zbuf, 8, 128)

        @pl.loop(0, rpt // 8)
        def _(i):
            pltpu.sync_copy(zbuf, acc_sh.at[pl.ds(s * rpt + i * 8, 8)])

        plsc.subcore_barrier()

        nmy = (n_chunks - wid + NC * NS - 1) // (NC * NS)

        @pl.loop(0, nmy)
        def _(i):
            base = (wid + i * NC * NS) * CHUNK
            pltpu.sync_copy(dst_hbm.at[pl.ds(base, CHUNK)], idx_v)
            pltpu.sync_copy(ones_v, acc_sh.at[idx_v], add=True)

        plsc.subcore_barrier()
        pltpu.sync_copy(
            acc_sh.at[pl.ds(s * rpt, rpt)],
            out_hbm.at[c, pl.ds(s * rpt, rpt)],
        )

    return deg_k


@functools.lru_cache(maxsize=None)
def _prop_kernel(E, N, D, dual):
    n_chunks = E // CHUNK
    Np = _pad_nodes(N)
    rpt = Np // NS

    @functools.partial(
        pl.kernel,
        mesh=_MESH,
        out_type=jax.ShapeDtypeStruct((NC, Np, D), jnp.float32),
        scratch_types=[
            pltpu.VMEM((CHUNK,), jnp.int32),
            pltpu.VMEM((CHUNK,), jnp.int32),
            pltpu.VMEM((CHUNK, D), jnp.float32),
            pltpu.VMEM((8, D), jnp.float32),
            pltpu.VMEM_SHARED((Np, D), jnp.float32),
            pltpu.SemaphoreType.DMA,
        ],
    )
    def prop_k(tab_hbm, src_hbm, dst_hbm, out_hbm, sidx, didx, rows, zbuf, acc_sh, sem):
        c = lax.axis_index("c")
        s = lax.axis_index("s")

        _zero_fill(zbuf, 8, D)

        @pl.loop(0, rpt // 8)
        def _(i):
            pltpu.sync_copy(zbuf, acc_sh.at[pl.ds(s * rpt + i * 8, 8)])

        plsc.subcore_barrier()

        if dual:
            wid = s
            nw = NS
            off = c * N
        else:
            wid = s * NC + c
            nw = NC * NS
            off = 0
        nmy = (n_chunks - wid + nw - 1) // nw

        @pl.loop(0, nmy)
        def _(i):
            base = (wid + i * nw) * CHUNK
            pltpu.sync_copy(src_hbm.at[pl.ds(base, CHUNK)], sidx)
            pltpu.sync_copy(dst_hbm.at[pl.ds(base, CHUNK)], didx)
            if dual:
                @pl.loop(0, CHUNK // 16)
                def _(j):
                    sidx[pl.ds(j * 16, 16)] = sidx[pl.ds(j * 16, 16)] + off
            pltpu.async_copy(tab_hbm.at[sidx], rows, sem).wait()
            pltpu.sync_copy(rows, acc_sh.at[didx], add=True)

        plsc.subcore_barrier()
        pltpu.sync_copy(
            acc_sh.at[pl.ds(s * rpt, rpt)],
            out_hbm.at[c, pl.ds(s * rpt, rpt)],
        )

    return prop_k


def kernel(x, edge_index, W1, b1, Wmu, bmu, Wlv, blv):
    N, D = x.shape
    E = edge_index.shape[1]
    src = edge_index[0]
    dst = edge_index[1]
    b1r = b1.reshape(1, D)
    bmur = bmu.reshape(1, D)
    blvr = blv.reshape(1, D)
    eps = jax.random.normal(jax.random.key(1), (N, D), jnp.float32)

    degp = _deg_kernel(E, N)(dst)

    def tcA(degp_r, x_r, W1_r, g1_o, dinv_o):
        deg = degp_r[0, :N, 0:1] + degp_r[1, :N, 0:1] + 1.0
        dinv = lax.rsqrt(deg)
        dinv_o[...] = dinv
        g1_o[...] = jnp.dot(x_r[...], W1_r[...], preferred_element_type=jnp.float32) * dinv

    g1, dinv = pl.pallas_call(
        tcA,
        out_shape=[
            jax.ShapeDtypeStruct((N, D), jnp.float32),
            jax.ShapeDtypeStruct((N, 1), jnp.float32),
        ],
    )(degp, x, W1)

    acc1 = _prop_kernel(E, N, D, False)(g1, src, dst)

    def tcB(acc_r, g1_r, dinv_r, b1_r, Wmu_r, Wlv_r, gml_o):
        h1 = jnp.maximum(
            (acc_r[0, :N, :] + acc_r[1, :N, :] + g1_r[...]) * dinv_r[...] + b1_r[...],
            0.0,
        )
        gml_o[0] = jnp.dot(h1, Wmu_r[...], preferred_element_type=jnp.float32) * dinv_r[...]
        gml_o[1] = jnp.dot(h1, Wlv_r[...], preferred_element_type=jnp.float32) * dinv_r[...]

    gml = pl.pallas_call(
        tcB, out_shape=jax.ShapeDtypeStruct((2, N, D), jnp.float32)
    )(acc1, g1, dinv, b1r, Wmu, Wlv)

    accml = _prop_kernel(E, N, D, True)(gml.reshape(2 * N, D), src, dst)

    def tcC(acc_r, gml_r, dinv_r, bmu_r, blv_r, eps_r, W1_r, mu_o, lv_o, g4_o):
        mu = (acc_r[0, :N, :] + gml_r[0]) * dinv_r[...] + bmu_r[...]
        lv = (acc_r[1, :N, :] + gml_r[1]) * dinv_r[...] + blv_r[...]
        mu_o[...] = mu
        lv_o[...] = lv
        z = eps_r[...] * jnp.exp(0.5 * lv) + mu
        g4_o[...] = jnp.dot(z, W1_r[...], preferred_element_type=jnp.float32) * dinv_r[...]

    mu, logvar, g4 = pl.pallas_call(
        tcC,
        out_shape=[jax.ShapeDtypeStruct((N, D), jnp.float32)] * 3,
    )(accml, gml, dinv, bmur, blvr, eps, W1)

    acc4 = _prop_kernel(E, N, D, False)(g4, src, dst)

    def tcD(acc_r, g4_r, dinv_r, b1_r, dec_o):
        pre = (acc_r[0, :N, :] + acc_r[1, :N, :] + g4_r[...]) * dinv_r[...] + b1_r[...]
        dec_o[...] = jax.nn.sigmoid(pre)

    dec = pl.pallas_call(
        tcD, out_shape=jax.ShapeDtypeStruct((N, D), jnp.float32)
    )(acc4, g4, dinv, b1r)

    return (dec, mu, logvar)
